# baseline (device time: 29812 ns/iter reference)
import jax
import jax.numpy as jnp
from jax import lax
from jax.experimental import pallas as pl
from jax.experimental.pallas import tpu as pltpu

B, SQ, H, D = 4, 32, 8, 128
KV_PER_SHARD = 4096
CK = 1024
N_CHUNKS = KV_PER_SHARD // CK
HG = 4
N_G = H // HG
LW = 8
SCALE = D ** -0.5


def kernel(Q, K, V):
    def body(q_ref, k_ref, v_ref, out_ref,
             qbuf, kbuf, vbuf, acc, lbuf, accsend, accr, lr, obuf,
             ksem, vsem, zs_acc, zr_acc, zs_l, zr_l,
             gsend_sems, grecv_sems):
        my_x = lax.axis_index("x")
        my_y = lax.axis_index("y")
        my_z = lax.axis_index("z")
        my_b = 2 * my_x + my_y

        for dz in (1, 2, 3):
            pl.semaphore_signal(
                pltpu.get_barrier_semaphore(), inc=1,
                device_id=(my_x, my_y, jnp.bitwise_xor(my_z, dz)),
                device_id_type=pl.DeviceIdType.MESH)
        for peer in ((1 - my_x, my_y, my_z), (my_x, 1 - my_y, my_z),
                     (1 - my_x, 1 - my_y, my_z)):
            pl.semaphore_signal(
                pltpu.get_barrier_semaphore(), inc=1, device_id=peer,
                device_id_type=pl.DeviceIdType.MESH)

        qbuf[...] = (q_ref[my_b] * SCALE).astype(jnp.bfloat16)
        acc[...] = jnp.zeros_like(acc)
        lbuf[...] = jnp.zeros_like(lbuf)

        def chunk_copies(t, slot):
            g, c = divmod(t, N_CHUNKS)
            copies = []
            for hl in range(HG):
                h = g * HG + hl
                copies.append(pltpu.make_async_copy(
                    k_ref.at[my_b, pl.ds(c * CK, CK), h, :],
                    kbuf.at[slot, hl], ksem.at[slot, hl]))
                copies.append(pltpu.make_async_copy(
                    v_ref.at[my_b, pl.ds(c * CK, CK), h, :],
                    vbuf.at[slot, hl], vsem.at[slot, hl]))
            return copies

        def start_dma(t, slot):
            for cp in chunk_copies(t, slot):
                cp.start()

        def z_rdmas(g):
            hs = pl.ds(g * HG, HG)
            rdmas = []
            for j, dz in enumerate((1, 2, 3)):
                peer = (my_x, my_y, jnp.bitwise_xor(my_z, dz))
                rdmas.append(pltpu.make_async_remote_copy(
                    src_ref=accsend.at[hs], dst_ref=accr.at[j, hs],
                    send_sem=zs_acc.at[g, j], recv_sem=zr_acc.at[g, j],
                    device_id=peer, device_id_type=pl.DeviceIdType.MESH))
                rdmas.append(pltpu.make_async_remote_copy(
                    src_ref=lbuf.at[hs], dst_ref=lr.at[j, hs],
                    send_sem=zs_l.at[g, j], recv_sem=zr_l.at[g, j],
                    device_id=peer, device_id_type=pl.DeviceIdType.MESH))
            return rdmas

        start_dma(0, 0)
        for g in range(N_G):
            for c in range(N_CHUNKS):
                t = g * N_CHUNKS + c
                slot = t % 2
                if t + 1 < N_G * N_CHUNKS:
                    start_dma(t + 1, (t + 1) % 2)
                for cp in chunk_copies(t, slot):
                    cp.wait()
                for hl in range(HG):
                    h = g * HG + hl
                    kb = kbuf[slot, hl].astype(jnp.bfloat16)
                    qb = qbuf[:, h, :]
                    s = lax.dot_general(
                        qb, kb, (((1,), (1,)), ((), ())),
                        preferred_element_type=jnp.float32)
                    p = jnp.exp(s.astype(jnp.bfloat16))
                    lsum = lax.dot_general(
                        p, jnp.ones((CK, LW), jnp.bfloat16),
                        (((1,), (0,)), ((), ())),
                        preferred_element_type=jnp.float32)
                    lbuf[h] = lbuf[h] + lsum
                    vb = vbuf[slot, hl].astype(jnp.bfloat16)
                    pv = lax.dot_general(
                        p, vb, (((1,), (0,)), ((), ())),
                        preferred_element_type=jnp.float32)
                    acc[h] = acc[h] + pv
            hs = pl.ds(g * HG, HG)
            accsend[hs] = acc[hs].astype(jnp.bfloat16)
            if g == 0:
                pl.semaphore_wait(pltpu.get_barrier_semaphore(), 6)
            for r in z_rdmas(g):
                r.start()

        xy_peers = ((1 - my_x, my_y, my_z), (my_x, 1 - my_y, my_z),
                    (1 - my_x, 1 - my_y, my_z))

        def gather_rdmas(g):
            hs = pl.ds(g * HG, HG)
            return [pltpu.make_async_remote_copy(
                src_ref=obuf.at[my_b, :, hs, :],
                dst_ref=obuf.at[my_b, :, hs, :],
                send_sem=gsend_sems.at[g, i], recv_sem=grecv_sems.at[g, i],
                device_id=peer, device_id_type=pl.DeviceIdType.MESH)
                for i, peer in enumerate(xy_peers)]

        for g in range(N_G):
            for r in z_rdmas(g):
                r.wait()
            hs = pl.ds(g * HG, HG)
            acc[hs] = (acc[hs]
                       + accr[0, hs].astype(jnp.float32)
                       + accr[1, hs].astype(jnp.float32)
                       + accr[2, hs].astype(jnp.float32))
            lbuf[hs] = lbuf[hs] + lr[0, hs] + lr[1, hs] + lr[2, hs]
            for hl in range(HG):
                h = g * HG + hl
                linv = jnp.broadcast_to(lbuf[h][:, 0:1], (SQ, D))
                obuf[my_b, :, h, :] = (acc[h] / linv).astype(jnp.bfloat16)
            for r in gather_rdmas(g):
                r.start()
        for g in range(N_G):
            for r in gather_rdmas(g):
                r.wait()

        out_ref[...] = obuf[...].astype(jnp.float32)

    return pl.pallas_call(
        body,
        out_shape=jax.ShapeDtypeStruct((B, SQ, H, D), jnp.float32),
        in_specs=[
            pl.BlockSpec(memory_space=pltpu.VMEM),
            pl.BlockSpec(memory_space=pl.ANY),
            pl.BlockSpec(memory_space=pl.ANY),
        ],
        out_specs=pl.BlockSpec(memory_space=pltpu.VMEM),
        scratch_shapes=[
            pltpu.VMEM((SQ, H, D), jnp.bfloat16),
            pltpu.VMEM((2, HG, CK, D), jnp.float32),
            pltpu.VMEM((2, HG, CK, D), jnp.float32),
            pltpu.VMEM((H, SQ, D), jnp.float32),
            pltpu.VMEM((H, SQ, LW), jnp.float32),
            pltpu.VMEM((H, SQ, D), jnp.bfloat16),
            pltpu.VMEM((3, H, SQ, D), jnp.bfloat16),
            pltpu.VMEM((3, H, SQ, LW), jnp.float32),
            pltpu.VMEM((B, SQ, H, D), jnp.bfloat16),
            pltpu.SemaphoreType.DMA((2, HG)),
            pltpu.SemaphoreType.DMA((2, HG)),
            pltpu.SemaphoreType.DMA((N_G, 3)),
            pltpu.SemaphoreType.DMA((N_G, 3)),
            pltpu.SemaphoreType.DMA((N_G, 3)),
            pltpu.SemaphoreType.DMA((N_G, 3)),
            pltpu.SemaphoreType.DMA((N_G, 3)),
            pltpu.SemaphoreType.DMA((N_G, 3)),
        ],
        compiler_params=pltpu.CompilerParams(collective_id=0),
    )(Q, K, V)


# device time: 29051 ns/iter; 1.0262x vs baseline; 1.0262x over previous
import jax
import jax.numpy as jnp
from jax import lax
from jax.experimental import pallas as pl
from jax.experimental.pallas import tpu as pltpu

B, SQ, H, D = 4, 32, 8, 128
KV_PER_SHARD = 4096
CK = 1024
N_CHUNKS = KV_PER_SHARD // CK
HG = 4
N_G = H // HG
LW = 8
SCALE = D ** -0.5


def kernel(Q, K, V):
    def body(q_ref, k_ref, v_ref, out_ref,
             qbuf, kbuf, vbuf, acc, lbuf, accsend, accr, lr, obuf,
             ksem, vsem, zs_acc, zr_acc, zs_l, zr_l,
             gsend_sems, grecv_sems):
        my_x = lax.axis_index("x")
        my_y = lax.axis_index("y")
        my_z = lax.axis_index("z")
        my_b = 2 * my_x + my_y

        for dz in (1, 2, 3):
            pl.semaphore_signal(
                pltpu.get_barrier_semaphore(), inc=1,
                device_id=(my_x, my_y, jnp.bitwise_xor(my_z, dz)),
                device_id_type=pl.DeviceIdType.MESH)
        for peer in ((1 - my_x, my_y, my_z), (my_x, 1 - my_y, my_z),
                     (1 - my_x, 1 - my_y, my_z)):
            pl.semaphore_signal(
                pltpu.get_barrier_semaphore(), inc=1, device_id=peer,
                device_id_type=pl.DeviceIdType.MESH)

        qbuf[...] = (q_ref[my_b] * SCALE).astype(jnp.bfloat16)
        acc[...] = jnp.zeros_like(acc)
        lbuf[...] = jnp.zeros_like(lbuf)

        def chunk_copies(t, slot):
            g, c = divmod(t, N_CHUNKS)
            copies = []
            for hl in range(HG):
                h = g * HG + hl
                copies.append(pltpu.make_async_copy(
                    k_ref.at[my_b, pl.ds(c * CK, CK), h, :],
                    kbuf.at[slot, hl], ksem.at[slot, hl]))
                copies.append(pltpu.make_async_copy(
                    v_ref.at[my_b, pl.ds(c * CK, CK), h, :],
                    vbuf.at[slot, hl], vsem.at[slot, hl]))
            return copies

        def start_dma(t, slot):
            for cp in chunk_copies(t, slot):
                cp.start()

        def z_rdmas(g):
            hs = pl.ds(g * HG, HG)
            rdmas = []
            for j, dz in enumerate((1, 2, 3)):
                peer = (my_x, my_y, jnp.bitwise_xor(my_z, dz))
                rdmas.append(pltpu.make_async_remote_copy(
                    src_ref=accsend.at[hs], dst_ref=accr.at[j, hs],
                    send_sem=zs_acc.at[g, j], recv_sem=zr_acc.at[g, j],
                    device_id=peer, device_id_type=pl.DeviceIdType.MESH))
                rdmas.append(pltpu.make_async_remote_copy(
                    src_ref=lbuf.at[hs], dst_ref=lr.at[j, hs],
                    send_sem=zs_l.at[g, j], recv_sem=zr_l.at[g, j],
                    device_id=peer, device_id_type=pl.DeviceIdType.MESH))
            return rdmas

        start_dma(0, 0)
        for g in range(N_G):
            for c in range(N_CHUNKS):
                t = g * N_CHUNKS + c
                slot = t % 2
                if t + 1 < N_G * N_CHUNKS:
                    start_dma(t + 1, (t + 1) % 2)
                for cp in chunk_copies(t, slot):
                    cp.wait()
                for hl in range(HG):
                    h = g * HG + hl
                    kb = kbuf[slot, hl].astype(jnp.bfloat16)
                    qb = qbuf[:, h, :]
                    s = lax.dot_general(
                        qb, kb, (((1,), (1,)), ((), ())),
                        preferred_element_type=jnp.float32)
                    p = jnp.exp(s)
                    lsum = jnp.sum(p, axis=1, keepdims=True)
                    lbuf[h] = lbuf[h] + jnp.broadcast_to(lsum, (SQ, LW))
                    vb = vbuf[slot, hl].astype(jnp.bfloat16)
                    pv = lax.dot_general(
                        p.astype(jnp.bfloat16), vb,
                        (((1,), (0,)), ((), ())),
                        preferred_element_type=jnp.float32)
                    acc[h] = acc[h] + pv
            hs = pl.ds(g * HG, HG)
            accsend[hs] = acc[hs].astype(jnp.bfloat16)
            if g == 0:
                pl.semaphore_wait(pltpu.get_barrier_semaphore(), 6)
            for r in z_rdmas(g):
                r.start()

        xy_peers = ((1 - my_x, my_y, my_z), (my_x, 1 - my_y, my_z),
                    (1 - my_x, 1 - my_y, my_z))

        def gather_rdmas(g):
            hs = pl.ds(g * HG, HG)
            return [pltpu.make_async_remote_copy(
                src_ref=obuf.at[my_b, :, hs, :],
                dst_ref=obuf.at[my_b, :, hs, :],
                send_sem=gsend_sems.at[g, i], recv_sem=grecv_sems.at[g, i],
                device_id=peer, device_id_type=pl.DeviceIdType.MESH)
                for i, peer in enumerate(xy_peers)]

        for g in range(N_G):
            for r in z_rdmas(g):
                r.wait()
            hs = pl.ds(g * HG, HG)
            acc[hs] = (acc[hs]
                       + accr[0, hs].astype(jnp.float32)
                       + accr[1, hs].astype(jnp.float32)
                       + accr[2, hs].astype(jnp.float32))
            lbuf[hs] = lbuf[hs] + lr[0, hs] + lr[1, hs] + lr[2, hs]
            for hl in range(HG):
                h = g * HG + hl
                linv = jnp.broadcast_to(lbuf[h][:, 0:1], (SQ, D))
                obuf[my_b, :, h, :] = (acc[h] / linv).astype(jnp.bfloat16)
            for r in gather_rdmas(g):
                r.start()
        for g in range(N_G):
            for r in gather_rdmas(g):
                r.wait()

        out_ref[...] = obuf[...].astype(jnp.float32)

    return pl.pallas_call(
        body,
        out_shape=jax.ShapeDtypeStruct((B, SQ, H, D), jnp.float32),
        in_specs=[
            pl.BlockSpec(memory_space=pltpu.VMEM),
            pl.BlockSpec(memory_space=pl.ANY),
            pl.BlockSpec(memory_space=pl.ANY),
        ],
        out_specs=pl.BlockSpec(memory_space=pltpu.VMEM),
        scratch_shapes=[
            pltpu.VMEM((SQ, H, D), jnp.bfloat16),
            pltpu.VMEM((2, HG, CK, D), jnp.float32),
            pltpu.VMEM((2, HG, CK, D), jnp.float32),
            pltpu.VMEM((H, SQ, D), jnp.float32),
            pltpu.VMEM((H, SQ, LW), jnp.float32),
            pltpu.VMEM((H, SQ, D), jnp.bfloat16),
            pltpu.VMEM((3, H, SQ, D), jnp.bfloat16),
            pltpu.VMEM((3, H, SQ, LW), jnp.float32),
            pltpu.VMEM((B, SQ, H, D), jnp.bfloat16),
            pltpu.SemaphoreType.DMA((2, HG)),
            pltpu.SemaphoreType.DMA((2, HG)),
            pltpu.SemaphoreType.DMA((N_G, 3)),
            pltpu.SemaphoreType.DMA((N_G, 3)),
            pltpu.SemaphoreType.DMA((N_G, 3)),
            pltpu.SemaphoreType.DMA((N_G, 3)),
            pltpu.SemaphoreType.DMA((N_G, 3)),
            pltpu.SemaphoreType.DMA((N_G, 3)),
        ],
        compiler_params=pltpu.CompilerParams(collective_id=0),
    )(Q, K, V)
